# baseline (speedup 1.0000x reference)
import jax
import jax.numpy as jnp
from jax.experimental import pallas as pl
from jax.experimental.pallas import tpu as pltpu
from jax.sharding import Mesh, PartitionSpec as P

D_MODEL = 1024
N_HEAD = 16
DH = 64
N_INTER = 4096
LORA_R = 8
LORA_SCALE = 2.0
N_EXPERTS = 4
B, S, L = 2, 2048, 256

_F32 = jnp.float32
_BF16 = jnp.bfloat16


def _norm_router_kernel(x_ref, ln_ref, wr_ref, br_ref, a1_ref,
                        nx_ref, comb_ref):
    x = x_ref[...]
    var = jnp.mean(x * x, axis=-1, keepdims=True)
    nx = x * jax.lax.rsqrt(var + 1e-6) * ln_ref[...][None]
    nx_ref[...] = nx.astype(_BF16)
    q_pool = jnp.mean(nx, axis=1)
    logits = jax.lax.dot_general(
        q_pool, wr_ref[...], (((1,), (0,)), ((), ())),
        preferred_element_type=_F32) + br_ref[...]
    aw = jax.nn.softmax(logits, axis=-1)
    idx = jax.lax.broadcasted_iota(jnp.int32, aw.shape, 1)
    big = jnp.int32(N_EXPERTS)
    w1 = jnp.max(aw, axis=-1, keepdims=True)
    i1 = jnp.min(jnp.where(aw >= w1, idx, big), axis=-1, keepdims=True)
    m = jnp.where(idx == i1, -jnp.inf, aw)
    w2 = jnp.max(m, axis=-1, keepdims=True)
    i2 = jnp.min(jnp.where(m >= w2, idx, big), axis=-1, keepdims=True)
    denom = w1 + w2 + 1e-10
    comb = jnp.where(idx == i1, w1, jnp.where(idx == i2, w2, 0.0)) / denom
    comb_ref[...] = comb * (1.0 / (1.0 + jnp.exp(-a1_ref[0, 0])))


def _norm_router(x, ln1_w, wr, br, a1):
    bl = x.shape[0]
    return pl.pallas_call(
        _norm_router_kernel,
        in_specs=[
            pl.BlockSpec((bl, S, D_MODEL), lambda: (0, 0, 0)),
            pl.BlockSpec((1, D_MODEL), lambda: (0, 0)),
            pl.BlockSpec((D_MODEL, N_EXPERTS), lambda: (0, 0)),
            pl.BlockSpec((1, N_EXPERTS), lambda: (0, 0)),
            pl.BlockSpec(memory_space=pltpu.SMEM),
        ],
        out_specs=(
            pl.BlockSpec((bl, S, D_MODEL), lambda: (0, 0, 0)),
            pl.BlockSpec((bl, N_EXPERTS), lambda: (0, 0)),
        ),
        out_shape=(
            jax.ShapeDtypeStruct((bl, S, D_MODEL), _BF16),
            jax.ShapeDtypeStruct((bl, N_EXPERTS), _F32),
        ),
    )(x, ln1_w.reshape(1, D_MODEL), wr, br.reshape(1, N_EXPERTS), a1)


_SB = 1024


def _attn_kernel(comb_ref, nx_ref, z_ref, wq_ref, wk_ref, wv_ref, wo_ref,
                 out_ref, q_s, k_s, v_s, o_s):
    b = pl.program_id(0)
    e = pl.program_id(1)

    @pl.when(e == 0)
    def _init():
        out_ref[...] = jnp.zeros_like(out_ref)

    w = comb_ref[b, e]

    @pl.when(w > 0.0)
    def _compute():
        z = z_ref[0, 0]
        k_s[...] = jnp.dot(z, wk_ref[0],
                           preferred_element_type=_F32).astype(_BF16)
        v_s[...] = jnp.dot(z, wv_ref[0],
                           preferred_element_type=_F32).astype(_BF16)
        for half in range(S // _SB):
            rows_g = slice(half * _SB, (half + 1) * _SB)
            q_s[...] = jnp.dot(nx_ref[0, rows_g], wq_ref[0],
                               preferred_element_type=_F32).astype(_BF16)
            for h in range(N_HEAD):
                cols = slice(h * DH, (h + 1) * DH)
                kh = k_s[:, cols]
                vh = v_s[:, cols]
                for sb in range(_SB // 512):
                    rows = slice(sb * 512, (sb + 1) * 512)
                    s = jax.lax.dot_general(
                        q_s[rows, cols], kh, (((1,), (1,)), ((), ())),
                        preferred_element_type=_F32) * 0.125
                    p = jax.nn.softmax(s, axis=-1).astype(_BF16)
                    o_s[rows, cols] = jnp.dot(
                        p, vh, preferred_element_type=_F32).astype(_BF16)
            out_ref[0, rows_g] += jnp.dot(
                o_s[...], wo_ref[0], preferred_element_type=_F32) * w


def _attn(comb, nx, zs, wqs, wks, wvs, wos):
    bl = nx.shape[0]
    wspec = pl.BlockSpec((1, D_MODEL, D_MODEL), lambda b, e: (e, 0, 0))
    return pl.pallas_call(
        _attn_kernel,
        grid=(bl, 3),
        in_specs=[
            pl.BlockSpec(memory_space=pltpu.SMEM),
            pl.BlockSpec((1, S, D_MODEL), lambda b, e: (b, 0, 0)),
            pl.BlockSpec((1, 1, L, D_MODEL), lambda b, e: (e, b, 0, 0)),
            wspec, wspec, wspec, wspec,
        ],
        out_specs=pl.BlockSpec((1, S, D_MODEL), lambda b, e: (b, 0, 0)),
        out_shape=jax.ShapeDtypeStruct((bl, S, D_MODEL), _F32),
        scratch_shapes=[
            pltpu.VMEM((_SB, D_MODEL), _BF16),
            pltpu.VMEM((L, D_MODEL), _BF16),
            pltpu.VMEM((L, D_MODEL), _BF16),
            pltpu.VMEM((_SB, D_MODEL), _BF16),
        ],
    )(comb, nx, zs, wqs, wks, wvs, wos)


_TB = 1024
_JB = 512
_NJ = N_INTER // _JB


def _mlp_kernel(x_ref, xm_ref, ln_ref, wg_ref, wu_ref, wd_ref,
                ag_ref, bg_ref, au_ref, bu_ref, ad_ref, bd_ref,
                a2_ref, out_ref,
                x1_s, h_s, lg_s, lu_s, acc_s, tl_s):
    j = pl.program_id(1)

    @pl.when(j == 0)
    def _prep():
        x1 = x_ref[...] + xm_ref[...]
        x1_s[...] = x1
        var = jnp.mean(x1 * x1, axis=-1, keepdims=True)
        h = x1 * jax.lax.rsqrt(var + 1e-6) * ln_ref[...]
        hb = h.astype(_BF16)
        h_s[...] = hb
        lg_s[...] = jnp.dot(hb, ag_ref[...],
                            preferred_element_type=_F32).astype(_BF16)
        lu_s[...] = jnp.dot(hb, au_ref[...],
                            preferred_element_type=_F32).astype(_BF16)
        acc_s[...] = jnp.zeros_like(acc_s)
        tl_s[...] = jnp.zeros_like(tl_s)

    hb = h_s[...]
    g = jnp.dot(hb, wg_ref[...], preferred_element_type=_F32)
    g += LORA_SCALE * jnp.dot(lg_s[...], bg_ref[...],
                              preferred_element_type=_F32)
    u = jnp.dot(hb, wu_ref[...], preferred_element_type=_F32)
    u += LORA_SCALE * jnp.dot(lu_s[...], bu_ref[...],
                              preferred_element_type=_F32)
    d = (g * jax.nn.sigmoid(g) + u).astype(_BF16)
    acc_s[...] += jnp.dot(d, wd_ref[...], preferred_element_type=_F32)
    tl_s[...] += jnp.dot(d, ad_ref[...], preferred_element_type=_F32)

    @pl.when(j == _NJ - 1)
    def _fin():
        mlp = acc_s[...] + LORA_SCALE * jnp.dot(
            tl_s[...].astype(_BF16), bd_ref[...], preferred_element_type=_F32)
        out_ref[...] = x1_s[...] + a2_ref[0, 0] * mlp


def _mlp(x2, xm2, ln2_w, wg, wu, wd, ag, bg, au, bu, ad, bd, a2):
    nt = x2.shape[0] // _TB
    return pl.pallas_call(
        _mlp_kernel,
        grid=(nt, _NJ),
        in_specs=[
            pl.BlockSpec((_TB, D_MODEL), lambda t, j: (t, 0)),
            pl.BlockSpec((_TB, D_MODEL), lambda t, j: (t, 0)),
            pl.BlockSpec((1, D_MODEL), lambda t, j: (0, 0)),
            pl.BlockSpec((D_MODEL, _JB), lambda t, j: (0, j)),
            pl.BlockSpec((D_MODEL, _JB), lambda t, j: (0, j)),
            pl.BlockSpec((_JB, D_MODEL), lambda t, j: (j, 0)),
            pl.BlockSpec((D_MODEL, LORA_R), lambda t, j: (0, 0)),
            pl.BlockSpec((LORA_R, _JB), lambda t, j: (0, j)),
            pl.BlockSpec((D_MODEL, LORA_R), lambda t, j: (0, 0)),
            pl.BlockSpec((LORA_R, _JB), lambda t, j: (0, j)),
            pl.BlockSpec((_JB, LORA_R), lambda t, j: (j, 0)),
            pl.BlockSpec((LORA_R, D_MODEL), lambda t, j: (0, 0)),
            pl.BlockSpec(memory_space=pltpu.SMEM),
        ],
        out_specs=pl.BlockSpec((_TB, D_MODEL), lambda t, j: (t, 0)),
        out_shape=jax.ShapeDtypeStruct((x2.shape[0], D_MODEL), _F32),
        scratch_shapes=[
            pltpu.VMEM((_TB, D_MODEL), _F32),
            pltpu.VMEM((_TB, D_MODEL), _BF16),
            pltpu.VMEM((_TB, LORA_R), _BF16),
            pltpu.VMEM((_TB, LORA_R), _BF16),
            pltpu.VMEM((_TB, D_MODEL), _F32),
            pltpu.VMEM((_TB, LORA_R), _F32),
        ],
    )(x2, xm2, ln2_w.reshape(1, D_MODEL), wg, wu, wd,
      ag, bg, au, bu, ad, bd, a2)


def _pipeline(x, zs, ln1_w, wr, br, a1, wqs, wks, wvs, wos,
              ln2_w, wg, wu, wd, ag, bg, au, bu, ad, bd, a2):
    nx, comb = _norm_router(x, ln1_w, wr, br, a1)
    xm = _attn(comb, nx, zs, wqs, wks, wvs, wos)
    tok = x.shape[0] * S
    out = _mlp(x.reshape(tok, D_MODEL), xm.reshape(tok, D_MODEL),
               ln2_w, wg, wu, wd, ag, bg, au, bu, ad, bd, a2)
    return out.reshape(x.shape)


def kernel(x_q, z_a, z_v, z_av, params):
    p = params
    x = x_q[0]

    zs = jnp.stack([z_a, z_v, z_av]).astype(_BF16)
    wqs = jnp.stack([p['Wq_a'], p['Wq_v'], p['Wq_av']]).astype(_BF16)
    wks = jnp.stack([p['Wk_a'], p['Wk_v'], p['Wk_av']]).astype(_BF16)
    wvs = jnp.stack([p['Wv_a'], p['Wv_v'], p['Wv_av']]).astype(_BF16)
    wos = jnp.stack([p['Wo_a'], p['Wo_v'], p['Wo_av']]).astype(_BF16)
    a1 = p['alpha_1'].reshape(1, 1)
    a2 = jax.nn.sigmoid(p['alpha_2']).reshape(1, 1)
    wmlp = [p['Wg'].astype(_BF16), p['Wu'].astype(_BF16),
            p['Wd'].astype(_BF16),
            p['Ag'].astype(_BF16), p['Bg'].astype(_BF16),
            p['Au'].astype(_BF16), p['Bu'].astype(_BF16),
            p['Ad'].astype(_BF16), p['Bd'].astype(_BF16)]

    devs = jax.devices()
    ndev = 2 if (len(devs) >= 2 and B % 2 == 0) else 1
    mesh = Mesh(devs[:ndev], ('b',))
    rep = tuple(P() for _ in range(19))
    sharded = jax.shard_map(
        _pipeline, mesh=mesh,
        in_specs=(P('b'), P(None, 'b')) + rep,
        out_specs=P('b'),
        check_vma=False)
    return sharded(x, zs, p['ln1_w'], p['Wr'], p['br'], a1,
                   wqs, wks, wvs, wos, p['ln2_w'], *wmlp, a2)


# speedup vs baseline: 1.9369x; 1.9369x over previous
import jax
import jax.numpy as jnp
from jax.experimental import pallas as pl
from jax.experimental.pallas import tpu as pltpu

D_MODEL = 1024
N_HEAD = 16
DH = 64
N_INTER = 4096
LORA_R = 8
LORA_SCALE = 2.0
N_EXPERTS = 4
B, S, L = 2, 2048, 256

_F32 = jnp.float32
_BF16 = jnp.bfloat16


def _norm_router_kernel(x_ref, ln_ref, wr_ref, br_ref, a1_ref,
                        nx_ref, comb_ref):
    x = x_ref[...]
    var = jnp.mean(x * x, axis=-1, keepdims=True)
    nx = x * jax.lax.rsqrt(var + 1e-6) * ln_ref[...][None]
    nx_ref[...] = nx.astype(_BF16)
    q_pool = jnp.mean(nx, axis=1)
    logits = jax.lax.dot_general(
        q_pool, wr_ref[...], (((1,), (0,)), ((), ())),
        preferred_element_type=_F32) + br_ref[...]
    aw = jax.nn.softmax(logits, axis=-1)
    idx = jax.lax.broadcasted_iota(jnp.int32, aw.shape, 1)
    big = jnp.int32(N_EXPERTS)
    w1 = jnp.max(aw, axis=-1, keepdims=True)
    i1 = jnp.min(jnp.where(aw >= w1, idx, big), axis=-1, keepdims=True)
    m = jnp.where(idx == i1, -jnp.inf, aw)
    w2 = jnp.max(m, axis=-1, keepdims=True)
    i2 = jnp.min(jnp.where(m >= w2, idx, big), axis=-1, keepdims=True)
    denom = w1 + w2 + 1e-10
    comb = jnp.where(idx == i1, w1, jnp.where(idx == i2, w2, 0.0)) / denom
    comb_ref[...] = comb * (1.0 / (1.0 + jnp.exp(-a1_ref[0, 0])))


def _norm_router(x, ln1_w, wr, br, a1):
    bl = x.shape[0]
    return pl.pallas_call(
        _norm_router_kernel,
        in_specs=[
            pl.BlockSpec((bl, S, D_MODEL), lambda: (0, 0, 0)),
            pl.BlockSpec((1, D_MODEL), lambda: (0, 0)),
            pl.BlockSpec((D_MODEL, N_EXPERTS), lambda: (0, 0)),
            pl.BlockSpec((1, N_EXPERTS), lambda: (0, 0)),
            pl.BlockSpec(memory_space=pltpu.SMEM),
        ],
        out_specs=(
            pl.BlockSpec((bl, S, D_MODEL), lambda: (0, 0, 0)),
            pl.BlockSpec((bl, N_EXPERTS), lambda: (0, 0)),
        ),
        out_shape=(
            jax.ShapeDtypeStruct((bl, S, D_MODEL), _BF16),
            jax.ShapeDtypeStruct((bl, N_EXPERTS), _F32),
        ),
    )(x, ln1_w.reshape(1, D_MODEL), wr, br.reshape(1, N_EXPERTS), a1)


_SB = 1024


def _attn_kernel(comb_ref, nx_ref, z_ref, wq_ref, wk_ref, wv_ref, wo_ref,
                 out_ref, q_s, k_s, v_s, o_s):
    b = pl.program_id(0)
    e = pl.program_id(1)

    @pl.when(e == 0)
    def _init():
        out_ref[...] = jnp.zeros_like(out_ref)

    w = comb_ref[b, e]

    @pl.when(w > 0.0)
    def _compute():
        z = z_ref[0, 0]
        k_s[...] = jnp.dot(z, wk_ref[0],
                           preferred_element_type=_F32).astype(_BF16)
        v_s[...] = jnp.dot(z, wv_ref[0],
                           preferred_element_type=_F32).astype(_BF16)
        for half in range(S // _SB):
            rows_g = slice(half * _SB, (half + 1) * _SB)
            q_s[...] = jnp.dot(nx_ref[0, rows_g], wq_ref[0],
                               preferred_element_type=_F32).astype(_BF16)
            for h in range(N_HEAD):
                cols = slice(h * DH, (h + 1) * DH)
                kh = k_s[:, cols]
                vh = v_s[:, cols]
                for sb in range(_SB // 512):
                    rows = slice(sb * 512, (sb + 1) * 512)
                    s = jax.lax.dot_general(
                        q_s[rows, cols], kh, (((1,), (1,)), ((), ())),
                        preferred_element_type=_F32) * 0.125
                    p = jax.nn.softmax(s, axis=-1).astype(_BF16)
                    o_s[rows, cols] = jnp.dot(
                        p, vh, preferred_element_type=_F32).astype(_BF16)
            out_ref[0, rows_g] += jnp.dot(
                o_s[...], wo_ref[0], preferred_element_type=_F32) * w


def _attn(comb, nx, zs, wqs, wks, wvs, wos):
    bl = nx.shape[0]
    wspec = pl.BlockSpec((1, D_MODEL, D_MODEL), lambda b, e: (e, 0, 0))
    return pl.pallas_call(
        _attn_kernel,
        grid=(bl, 3),
        in_specs=[
            pl.BlockSpec(memory_space=pltpu.SMEM),
            pl.BlockSpec((1, S, D_MODEL), lambda b, e: (b, 0, 0)),
            pl.BlockSpec((1, 1, L, D_MODEL), lambda b, e: (e, b, 0, 0)),
            wspec, wspec, wspec, wspec,
        ],
        out_specs=pl.BlockSpec((1, S, D_MODEL), lambda b, e: (b, 0, 0)),
        out_shape=jax.ShapeDtypeStruct((bl, S, D_MODEL), _F32),
        scratch_shapes=[
            pltpu.VMEM((_SB, D_MODEL), _BF16),
            pltpu.VMEM((L, D_MODEL), _BF16),
            pltpu.VMEM((L, D_MODEL), _BF16),
            pltpu.VMEM((_SB, D_MODEL), _BF16),
        ],
    )(comb, nx, zs, wqs, wks, wvs, wos)


_TB = 1024
_JB = 512
_NJ = N_INTER // _JB


def _mlp_kernel(x_ref, xm_ref, ln_ref, wg_ref, wu_ref, wd_ref,
                ag_ref, bg_ref, au_ref, bu_ref, ad_ref, bd_ref,
                a2_ref, out_ref,
                x1_s, h_s, lg_s, lu_s, acc_s, tl_s):
    j = pl.program_id(1)

    @pl.when(j == 0)
    def _prep():
        x1 = x_ref[...] + xm_ref[...]
        x1_s[...] = x1
        var = jnp.mean(x1 * x1, axis=-1, keepdims=True)
        h = x1 * jax.lax.rsqrt(var + 1e-6) * ln_ref[...]
        hb = h.astype(_BF16)
        h_s[...] = hb
        lg_s[...] = jnp.dot(hb, ag_ref[...],
                            preferred_element_type=_F32).astype(_BF16)
        lu_s[...] = jnp.dot(hb, au_ref[...],
                            preferred_element_type=_F32).astype(_BF16)
        acc_s[...] = jnp.zeros_like(acc_s)
        tl_s[...] = jnp.zeros_like(tl_s)

    hb = h_s[...]
    g = jnp.dot(hb, wg_ref[...], preferred_element_type=_F32)
    g += LORA_SCALE * jnp.dot(lg_s[...], bg_ref[...],
                              preferred_element_type=_F32)
    u = jnp.dot(hb, wu_ref[...], preferred_element_type=_F32)
    u += LORA_SCALE * jnp.dot(lu_s[...], bu_ref[...],
                              preferred_element_type=_F32)
    d = (g * jax.nn.sigmoid(g) + u).astype(_BF16)
    acc_s[...] += jnp.dot(d, wd_ref[...], preferred_element_type=_F32)
    tl_s[...] += jnp.dot(d, ad_ref[...], preferred_element_type=_F32)

    @pl.when(j == _NJ - 1)
    def _fin():
        mlp = acc_s[...] + LORA_SCALE * jnp.dot(
            tl_s[...].astype(_BF16), bd_ref[...], preferred_element_type=_F32)
        out_ref[...] = x1_s[...] + a2_ref[0, 0] * mlp


def _mlp(x2, xm2, ln2_w, wg, wu, wd, ag, bg, au, bu, ad, bd, a2):
    nt = x2.shape[0] // _TB
    return pl.pallas_call(
        _mlp_kernel,
        grid=(nt, _NJ),
        in_specs=[
            pl.BlockSpec((_TB, D_MODEL), lambda t, j: (t, 0)),
            pl.BlockSpec((_TB, D_MODEL), lambda t, j: (t, 0)),
            pl.BlockSpec((1, D_MODEL), lambda t, j: (0, 0)),
            pl.BlockSpec((D_MODEL, _JB), lambda t, j: (0, j)),
            pl.BlockSpec((D_MODEL, _JB), lambda t, j: (0, j)),
            pl.BlockSpec((_JB, D_MODEL), lambda t, j: (j, 0)),
            pl.BlockSpec((D_MODEL, LORA_R), lambda t, j: (0, 0)),
            pl.BlockSpec((LORA_R, _JB), lambda t, j: (0, j)),
            pl.BlockSpec((D_MODEL, LORA_R), lambda t, j: (0, 0)),
            pl.BlockSpec((LORA_R, _JB), lambda t, j: (0, j)),
            pl.BlockSpec((_JB, LORA_R), lambda t, j: (j, 0)),
            pl.BlockSpec((LORA_R, D_MODEL), lambda t, j: (0, 0)),
            pl.BlockSpec(memory_space=pltpu.SMEM),
        ],
        out_specs=pl.BlockSpec((_TB, D_MODEL), lambda t, j: (t, 0)),
        out_shape=jax.ShapeDtypeStruct((x2.shape[0], D_MODEL), _F32),
        scratch_shapes=[
            pltpu.VMEM((_TB, D_MODEL), _F32),
            pltpu.VMEM((_TB, D_MODEL), _BF16),
            pltpu.VMEM((_TB, LORA_R), _BF16),
            pltpu.VMEM((_TB, LORA_R), _BF16),
            pltpu.VMEM((_TB, D_MODEL), _F32),
            pltpu.VMEM((_TB, LORA_R), _F32),
        ],
    )(x2, xm2, ln2_w.reshape(1, D_MODEL), wg, wu, wd,
      ag, bg, au, bu, ad, bd, a2)


def _pipeline(x, zs, ln1_w, wr, br, a1, wqs, wks, wvs, wos,
              ln2_w, wg, wu, wd, ag, bg, au, bu, ad, bd, a2):
    nx, comb = _norm_router(x, ln1_w, wr, br, a1)
    xm = _attn(comb, nx, zs, wqs, wks, wvs, wos)
    tok = x.shape[0] * S
    out = _mlp(x.reshape(tok, D_MODEL), xm.reshape(tok, D_MODEL),
               ln2_w, wg, wu, wd, ag, bg, au, bu, ad, bd, a2)
    return out.reshape(x.shape)


def kernel(x_q, z_a, z_v, z_av, params):
    p = params
    x = x_q[0]

    zs = jnp.stack([z_a, z_v, z_av]).astype(_BF16)
    wqs = jnp.stack([p['Wq_a'], p['Wq_v'], p['Wq_av']]).astype(_BF16)
    wks = jnp.stack([p['Wk_a'], p['Wk_v'], p['Wk_av']]).astype(_BF16)
    wvs = jnp.stack([p['Wv_a'], p['Wv_v'], p['Wv_av']]).astype(_BF16)
    wos = jnp.stack([p['Wo_a'], p['Wo_v'], p['Wo_av']]).astype(_BF16)
    a1 = p['alpha_1'].reshape(1, 1)
    a2 = jax.nn.sigmoid(p['alpha_2']).reshape(1, 1)
    wmlp = [p['Wg'].astype(_BF16), p['Wu'].astype(_BF16),
            p['Wd'].astype(_BF16),
            p['Ag'].astype(_BF16), p['Bg'].astype(_BF16),
            p['Au'].astype(_BF16), p['Bu'].astype(_BF16),
            p['Ad'].astype(_BF16), p['Bd'].astype(_BF16)]

    return _pipeline(x, zs, p['ln1_w'], p['Wr'], p['br'], a1,
                     wqs, wks, wvs, wos, p['ln2_w'], *wmlp, a2)


# speedup vs baseline: 2.0755x; 1.0716x over previous
import jax
import jax.numpy as jnp
from jax.experimental import pallas as pl
from jax.experimental.pallas import tpu as pltpu

D_MODEL = 1024
N_HEAD = 16
DH = 64
N_INTER = 4096
LORA_R = 8
LORA_SCALE = 2.0
N_EXPERTS = 4
B, S, L = 2, 2048, 256

_F32 = jnp.float32
_BF16 = jnp.bfloat16


def _norm_router_kernel(x_ref, ln_ref, wr_ref, br_ref, a1_ref,
                        nx_ref, comb_ref):
    x = x_ref[...]
    var = jnp.mean(x * x, axis=-1, keepdims=True)
    nx = x * jax.lax.rsqrt(var + 1e-6) * ln_ref[...][None]
    nx_ref[...] = nx.astype(_BF16)
    q_pool = jnp.mean(nx, axis=1)
    logits = jax.lax.dot_general(
        q_pool, wr_ref[...], (((1,), (0,)), ((), ())),
        preferred_element_type=_F32) + br_ref[...]
    aw = jax.nn.softmax(logits, axis=-1)
    idx = jax.lax.broadcasted_iota(jnp.int32, aw.shape, 1)
    big = jnp.int32(N_EXPERTS)
    w1 = jnp.max(aw, axis=-1, keepdims=True)
    i1 = jnp.min(jnp.where(aw >= w1, idx, big), axis=-1, keepdims=True)
    m = jnp.where(idx == i1, -jnp.inf, aw)
    w2 = jnp.max(m, axis=-1, keepdims=True)
    i2 = jnp.min(jnp.where(m >= w2, idx, big), axis=-1, keepdims=True)
    denom = w1 + w2 + 1e-10
    comb = jnp.where(idx == i1, w1, jnp.where(idx == i2, w2, 0.0)) / denom
    comb_ref[...] = comb * (1.0 / (1.0 + jnp.exp(-a1_ref[0, 0])))


def _norm_router(x, ln1_w, wr, br, a1):
    bl = x.shape[0]
    return pl.pallas_call(
        _norm_router_kernel,
        in_specs=[
            pl.BlockSpec((bl, S, D_MODEL), lambda: (0, 0, 0)),
            pl.BlockSpec((1, D_MODEL), lambda: (0, 0)),
            pl.BlockSpec((D_MODEL, N_EXPERTS), lambda: (0, 0)),
            pl.BlockSpec((1, N_EXPERTS), lambda: (0, 0)),
            pl.BlockSpec(memory_space=pltpu.SMEM),
        ],
        out_specs=(
            pl.BlockSpec((bl, S, D_MODEL), lambda: (0, 0, 0)),
            pl.BlockSpec((bl, N_EXPERTS), lambda: (0, 0)),
        ),
        out_shape=(
            jax.ShapeDtypeStruct((bl, S, D_MODEL), _BF16),
            jax.ShapeDtypeStruct((bl, N_EXPERTS), _F32),
        ),
    )(x, ln1_w.reshape(1, D_MODEL), wr, br.reshape(1, N_EXPERTS), a1)


_SB = 1024


def _attn_kernel(comb_ref, nx_ref,
                 za_ref, zv_ref, zav_ref,
                 wqa_ref, wka_ref, wva_ref, woa_ref,
                 wqv_ref, wkv_ref, wvv_ref, wov_ref,
                 wqav_ref, wkav_ref, wvav_ref, woav_ref,
                 out_ref,
                 z_st, wq_st, wk_st, wv_st, wo_st, q_s, k_s, v_s, o_s,
                 sem_z, sem_q, sem_k, sem_v, sem_o):
    b = pl.program_id(0)
    e = pl.program_id(1)

    @pl.when(e == 0)
    def _init():
        out_ref[...] = jnp.zeros_like(out_ref)

    w = comb_ref[b, e]

    @pl.when(w > 0.0)
    def _compute():
        def _start(zr, wq, wk, wv, wo):
            pltpu.make_async_copy(zr.at[b], z_st, sem_z).start()
            pltpu.make_async_copy(wk, wk_st, sem_k).start()
            pltpu.make_async_copy(wv, wv_st, sem_v).start()
            pltpu.make_async_copy(wq, wq_st, sem_q).start()
            pltpu.make_async_copy(wo, wo_st, sem_o).start()

        @pl.when(e == 0)
        def _sa():
            _start(za_ref, wqa_ref, wka_ref, wva_ref, woa_ref)

        @pl.when(e == 1)
        def _sv():
            _start(zv_ref, wqv_ref, wkv_ref, wvv_ref, wov_ref)

        @pl.when(e == 2)
        def _sav():
            _start(zav_ref, wqav_ref, wkav_ref, wvav_ref, woav_ref)

        pltpu.make_async_copy(za_ref.at[b], z_st, sem_z).wait()
        z = z_st[...].astype(_BF16)
        pltpu.make_async_copy(wka_ref, wk_st, sem_k).wait()
        k_s[...] = jnp.dot(z, wk_st[...].astype(_BF16),
                           preferred_element_type=_F32).astype(_BF16)
        pltpu.make_async_copy(wva_ref, wv_st, sem_v).wait()
        v_s[...] = jnp.dot(z, wv_st[...].astype(_BF16),
                           preferred_element_type=_F32).astype(_BF16)
        pltpu.make_async_copy(wqa_ref, wq_st, sem_q).wait()
        wqb = wq_st[...].astype(_BF16)
        pltpu.make_async_copy(woa_ref, wo_st, sem_o).wait()
        wob = wo_st[...].astype(_BF16)
        for half in range(S // _SB):
            rows_g = slice(half * _SB, (half + 1) * _SB)
            q_s[...] = jnp.dot(nx_ref[0, rows_g], wqb,
                               preferred_element_type=_F32).astype(_BF16)
            for h in range(N_HEAD):
                cols = slice(h * DH, (h + 1) * DH)
                kh = k_s[:, cols]
                vh = v_s[:, cols]
                for sb in range(_SB // 512):
                    rows = slice(sb * 512, (sb + 1) * 512)
                    s = jax.lax.dot_general(
                        q_s[rows, cols], kh, (((1,), (1,)), ((), ())),
                        preferred_element_type=_F32) * 0.125
                    p = jax.nn.softmax(s, axis=-1).astype(_BF16)
                    o_s[rows, cols] = jnp.dot(
                        p, vh, preferred_element_type=_F32).astype(_BF16)
            out_ref[0, rows_g] += jnp.dot(
                o_s[...], wob, preferred_element_type=_F32) * w


def _attn(comb, nx, z_a, z_v, z_av, wlist):
    bl = nx.shape[0]
    anyspec = pl.BlockSpec(memory_space=pl.MemorySpace.ANY)
    return pl.pallas_call(
        _attn_kernel,
        grid=(bl, 3),
        in_specs=[
            pl.BlockSpec(memory_space=pltpu.SMEM),
            pl.BlockSpec((1, S, D_MODEL), lambda b, e: (b, 0, 0)),
        ] + [anyspec] * 15,
        out_specs=pl.BlockSpec((1, S, D_MODEL), lambda b, e: (b, 0, 0)),
        out_shape=jax.ShapeDtypeStruct((bl, S, D_MODEL), _F32),
        scratch_shapes=[
            pltpu.VMEM((L, D_MODEL), _F32),
            pltpu.VMEM((D_MODEL, D_MODEL), _F32),
            pltpu.VMEM((D_MODEL, D_MODEL), _F32),
            pltpu.VMEM((D_MODEL, D_MODEL), _F32),
            pltpu.VMEM((D_MODEL, D_MODEL), _F32),
            pltpu.VMEM((_SB, D_MODEL), _BF16),
            pltpu.VMEM((L, D_MODEL), _BF16),
            pltpu.VMEM((L, D_MODEL), _BF16),
            pltpu.VMEM((_SB, D_MODEL), _BF16),
            pltpu.SemaphoreType.DMA,
            pltpu.SemaphoreType.DMA,
            pltpu.SemaphoreType.DMA,
            pltpu.SemaphoreType.DMA,
            pltpu.SemaphoreType.DMA,
        ],
    )(comb, nx, z_a, z_v, z_av, *wlist)


_TB = 1024
_JB = 512
_NJ = N_INTER // _JB


def _mlp_kernel(x_ref, xm_ref, ln_ref, wg_ref, wu_ref, wd_ref,
                ag_ref, bg_ref, au_ref, bu_ref, ad_ref, bd_ref,
                a2_ref, out_ref,
                x1_s, h_s, lg_s, lu_s, acc_s, tl_s):
    j = pl.program_id(1)

    @pl.when(j == 0)
    def _prep():
        x1 = x_ref[...] + xm_ref[...]
        x1_s[...] = x1
        var = jnp.mean(x1 * x1, axis=-1, keepdims=True)
        h = x1 * jax.lax.rsqrt(var + 1e-6) * ln_ref[...]
        hb = h.astype(_BF16)
        h_s[...] = hb
        lg_s[...] = jnp.dot(hb, ag_ref[...],
                            preferred_element_type=_F32).astype(_BF16)
        lu_s[...] = jnp.dot(hb, au_ref[...],
                            preferred_element_type=_F32).astype(_BF16)
        acc_s[...] = jnp.zeros_like(acc_s)
        tl_s[...] = jnp.zeros_like(tl_s)

    hb = h_s[...]
    g = jnp.dot(hb, wg_ref[...], preferred_element_type=_F32)
    g += LORA_SCALE * jnp.dot(lg_s[...], bg_ref[...],
                              preferred_element_type=_F32)
    u = jnp.dot(hb, wu_ref[...], preferred_element_type=_F32)
    u += LORA_SCALE * jnp.dot(lu_s[...], bu_ref[...],
                              preferred_element_type=_F32)
    d = (g * jax.nn.sigmoid(g) + u).astype(_BF16)
    acc_s[...] += jnp.dot(d, wd_ref[...], preferred_element_type=_F32)
    tl_s[...] += jnp.dot(d, ad_ref[...], preferred_element_type=_F32)

    @pl.when(j == _NJ - 1)
    def _fin():
        mlp = acc_s[...] + LORA_SCALE * jnp.dot(
            tl_s[...].astype(_BF16), bd_ref[...], preferred_element_type=_F32)
        out_ref[...] = x1_s[...] + a2_ref[0, 0] * mlp


def _mlp(x2, xm2, ln2_w, wg, wu, wd, ag, bg, au, bu, ad, bd, a2):
    nt = x2.shape[0] // _TB
    return pl.pallas_call(
        _mlp_kernel,
        grid=(nt, _NJ),
        in_specs=[
            pl.BlockSpec((_TB, D_MODEL), lambda t, j: (t, 0)),
            pl.BlockSpec((_TB, D_MODEL), lambda t, j: (t, 0)),
            pl.BlockSpec((1, D_MODEL), lambda t, j: (0, 0)),
            pl.BlockSpec((D_MODEL, _JB), lambda t, j: (0, j)),
            pl.BlockSpec((D_MODEL, _JB), lambda t, j: (0, j)),
            pl.BlockSpec((_JB, D_MODEL), lambda t, j: (j, 0)),
            pl.BlockSpec((D_MODEL, LORA_R), lambda t, j: (0, 0)),
            pl.BlockSpec((LORA_R, _JB), lambda t, j: (0, j)),
            pl.BlockSpec((D_MODEL, LORA_R), lambda t, j: (0, 0)),
            pl.BlockSpec((LORA_R, _JB), lambda t, j: (0, j)),
            pl.BlockSpec((_JB, LORA_R), lambda t, j: (j, 0)),
            pl.BlockSpec((LORA_R, D_MODEL), lambda t, j: (0, 0)),
            pl.BlockSpec(memory_space=pltpu.SMEM),
        ],
        out_specs=pl.BlockSpec((_TB, D_MODEL), lambda t, j: (t, 0)),
        out_shape=jax.ShapeDtypeStruct((x2.shape[0], D_MODEL), _F32),
        scratch_shapes=[
            pltpu.VMEM((_TB, D_MODEL), _F32),
            pltpu.VMEM((_TB, D_MODEL), _BF16),
            pltpu.VMEM((_TB, LORA_R), _BF16),
            pltpu.VMEM((_TB, LORA_R), _BF16),
            pltpu.VMEM((_TB, D_MODEL), _F32),
            pltpu.VMEM((_TB, LORA_R), _F32),
        ],
    )(x2, xm2, ln2_w.reshape(1, D_MODEL), wg, wu, wd,
      ag, bg, au, bu, ad, bd, a2)


def kernel(x_q, z_a, z_v, z_av, params):
    p = params
    x = x_q[0]

    a1 = p['alpha_1'].reshape(1, 1)
    a2 = jax.nn.sigmoid(p['alpha_2']).reshape(1, 1)
    wattn = [p['Wq_a'], p['Wk_a'], p['Wv_a'], p['Wo_a'],
             p['Wq_v'], p['Wk_v'], p['Wv_v'], p['Wo_v'],
             p['Wq_av'], p['Wk_av'], p['Wv_av'], p['Wo_av']]
    wmlp = [p['Wg'].astype(_BF16), p['Wu'].astype(_BF16),
            p['Wd'].astype(_BF16),
            p['Ag'].astype(_BF16), p['Bg'].astype(_BF16),
            p['Au'].astype(_BF16), p['Bu'].astype(_BF16),
            p['Ad'].astype(_BF16), p['Bd'].astype(_BF16)]

    nx, comb = _norm_router(x, p['ln1_w'], p['Wr'], p['br'], a1)
    xm = _attn(comb, nx, z_a, z_v, z_av, wattn)
    tok = B * S
    out = _mlp(x.reshape(tok, D_MODEL), xm.reshape(tok, D_MODEL),
               p['ln2_w'], *wmlp, a2)
    return out.reshape(B, S, D_MODEL)


# speedup vs baseline: 2.2195x; 1.0694x over previous
import jax
import jax.numpy as jnp
from jax.experimental import pallas as pl
from jax.experimental.pallas import tpu as pltpu

D_MODEL = 1024
N_HEAD = 16
DH = 64
N_INTER = 4096
LORA_R = 8
LORA_SCALE = 2.0
N_EXPERTS = 4
B, S, L = 2, 2048, 256

_F32 = jnp.float32
_BF16 = jnp.bfloat16


def _norm_router_kernel(x_ref, ln_ref, wr_ref, br_ref, a1_ref,
                        nx_ref, comb_ref):
    x = x_ref[...]
    var = jnp.mean(x * x, axis=-1, keepdims=True)
    nx = x * jax.lax.rsqrt(var + 1e-6) * ln_ref[...][None]
    nx_ref[...] = nx.astype(_BF16)
    q_pool = jnp.mean(nx, axis=1)
    logits = jax.lax.dot_general(
        q_pool, wr_ref[...], (((1,), (0,)), ((), ())),
        preferred_element_type=_F32) + br_ref[...]
    aw = jax.nn.softmax(logits, axis=-1)
    idx = jax.lax.broadcasted_iota(jnp.int32, aw.shape, 1)
    big = jnp.int32(N_EXPERTS)
    w1 = jnp.max(aw, axis=-1, keepdims=True)
    i1 = jnp.min(jnp.where(aw >= w1, idx, big), axis=-1, keepdims=True)
    m = jnp.where(idx == i1, -jnp.inf, aw)
    w2 = jnp.max(m, axis=-1, keepdims=True)
    i2 = jnp.min(jnp.where(m >= w2, idx, big), axis=-1, keepdims=True)
    denom = w1 + w2 + 1e-10
    comb = jnp.where(idx == i1, w1, jnp.where(idx == i2, w2, 0.0)) / denom
    comb_ref[...] = comb * (1.0 / (1.0 + jnp.exp(-a1_ref[0, 0])))


def _norm_router(x, ln1_w, wr, br, a1):
    bl = x.shape[0]
    return pl.pallas_call(
        _norm_router_kernel,
        in_specs=[
            pl.BlockSpec((bl, S, D_MODEL), lambda: (0, 0, 0)),
            pl.BlockSpec((1, D_MODEL), lambda: (0, 0)),
            pl.BlockSpec((D_MODEL, N_EXPERTS), lambda: (0, 0)),
            pl.BlockSpec((1, N_EXPERTS), lambda: (0, 0)),
            pl.BlockSpec(memory_space=pltpu.SMEM),
        ],
        out_specs=(
            pl.BlockSpec((bl, S, D_MODEL), lambda: (0, 0, 0)),
            pl.BlockSpec((bl, N_EXPERTS), lambda: (0, 0)),
        ),
        out_shape=(
            jax.ShapeDtypeStruct((bl, S, D_MODEL), _BF16),
            jax.ShapeDtypeStruct((bl, N_EXPERTS), _F32),
        ),
    )(x, ln1_w.reshape(1, D_MODEL), wr, br.reshape(1, N_EXPERTS), a1)


_SB = 1024


def _attn_kernel(comb_ref, nx_ref,
                 za_ref, zv_ref, zav_ref,
                 wqa_ref, wka_ref, wva_ref, woa_ref,
                 wqv_ref, wkv_ref, wvv_ref, wov_ref,
                 wqav_ref, wkav_ref, wvav_ref, woav_ref,
                 out_ref,
                 z_st, wq_st, wk_st, wv_st, wo_st, q_s, k_s, v_s, o_s,
                 sem_z, sem_q, sem_k, sem_v, sem_o):
    b = pl.program_id(0)
    e = pl.program_id(1)

    @pl.when(e == 0)
    def _init():
        out_ref[...] = jnp.zeros_like(out_ref)

    w = comb_ref[b, e]

    @pl.when(w > 0.0)
    def _compute():
        def _start(zr, wq, wk, wv, wo):
            pltpu.make_async_copy(wq, wq_st, sem_q).start()
            pltpu.make_async_copy(zr.at[b], z_st, sem_z).start()
            pltpu.make_async_copy(wk, wk_st, sem_k).start()
            pltpu.make_async_copy(wv, wv_st, sem_v).start()
            pltpu.make_async_copy(wo, wo_st, sem_o).start()

        @pl.when(e == 0)
        def _sa():
            _start(za_ref, wqa_ref, wka_ref, wva_ref, woa_ref)

        @pl.when(e == 1)
        def _sv():
            _start(zv_ref, wqv_ref, wkv_ref, wvv_ref, wov_ref)

        @pl.when(e == 2)
        def _sav():
            _start(zav_ref, wqav_ref, wkav_ref, wvav_ref, woav_ref)

        pltpu.make_async_copy(wqa_ref, wq_st, sem_q).wait()
        wqb = wq_st[...].astype(_BF16)
        q_s[...] = jnp.dot(nx_ref[0, :_SB], wqb,
                           preferred_element_type=_F32).astype(_BF16)
        pltpu.make_async_copy(za_ref.at[b], z_st, sem_z).wait()
        z = z_st[...].astype(_BF16)
        pltpu.make_async_copy(wka_ref, wk_st, sem_k).wait()
        k_s[...] = jnp.dot(z, wk_st[...].astype(_BF16),
                           preferred_element_type=_F32).astype(_BF16)
        pltpu.make_async_copy(wva_ref, wv_st, sem_v).wait()
        v_s[...] = jnp.dot(z, wv_st[...].astype(_BF16),
                           preferred_element_type=_F32).astype(_BF16)
        pltpu.make_async_copy(woa_ref, wo_st, sem_o).wait()
        wob = wo_st[...].astype(_BF16)
        for half in range(S // _SB):
            rows_g = slice(half * _SB, (half + 1) * _SB)
            if half > 0:
                q_s[...] = jnp.dot(nx_ref[0, rows_g], wqb,
                                   preferred_element_type=_F32).astype(_BF16)
            for h in range(N_HEAD):
                cols = slice(h * DH, (h + 1) * DH)
                kh = k_s[:, cols]
                vh = v_s[:, cols]
                for sb in range(_SB // 512):
                    rows = slice(sb * 512, (sb + 1) * 512)
                    s = jax.lax.dot_general(
                        q_s[rows, cols], kh, (((1,), (1,)), ((), ())),
                        preferred_element_type=_F32) * 0.125
                    p = jax.nn.softmax(s, axis=-1).astype(_BF16)
                    o_s[rows, cols] = jnp.dot(
                        p, vh, preferred_element_type=_F32).astype(_BF16)
            out_ref[0, rows_g] += jnp.dot(
                o_s[...], wob, preferred_element_type=_F32) * w


def _attn(comb, nx, z_a, z_v, z_av, wlist):
    bl = nx.shape[0]
    anyspec = pl.BlockSpec(memory_space=pl.MemorySpace.ANY)
    return pl.pallas_call(
        _attn_kernel,
        grid=(bl, 3),
        in_specs=[
            pl.BlockSpec(memory_space=pltpu.SMEM),
            pl.BlockSpec((1, S, D_MODEL), lambda b, e: (b, 0, 0)),
        ] + [anyspec] * 15,
        out_specs=pl.BlockSpec((1, S, D_MODEL), lambda b, e: (b, 0, 0)),
        out_shape=jax.ShapeDtypeStruct((bl, S, D_MODEL), _F32),
        scratch_shapes=[
            pltpu.VMEM((L, D_MODEL), _F32),
            pltpu.VMEM((D_MODEL, D_MODEL), _F32),
            pltpu.VMEM((D_MODEL, D_MODEL), _F32),
            pltpu.VMEM((D_MODEL, D_MODEL), _F32),
            pltpu.VMEM((D_MODEL, D_MODEL), _F32),
            pltpu.VMEM((_SB, D_MODEL), _BF16),
            pltpu.VMEM((L, D_MODEL), _BF16),
            pltpu.VMEM((L, D_MODEL), _BF16),
            pltpu.VMEM((_SB, D_MODEL), _BF16),
            pltpu.SemaphoreType.DMA,
            pltpu.SemaphoreType.DMA,
            pltpu.SemaphoreType.DMA,
            pltpu.SemaphoreType.DMA,
            pltpu.SemaphoreType.DMA,
        ],
    )(comb, nx, z_a, z_v, z_av, *wlist)


_TB = 1024
_JB = 512
_NJ = N_INTER // _JB
_PC = 512


def _fold_kernel(w_ref, a_ref, b_ref, out_ref):
    out_ref[...] = (w_ref[...] + LORA_SCALE * jnp.dot(
        a_ref[...].astype(_BF16), b_ref[...].astype(_BF16),
        preferred_element_type=_F32)).astype(_BF16)


def _fold(w, a, bm):
    rows, cols = w.shape
    return pl.pallas_call(
        _fold_kernel,
        grid=(rows // _PC,),
        in_specs=[
            pl.BlockSpec((_PC, cols), lambda r: (r, 0)),
            pl.BlockSpec((_PC, LORA_R), lambda r: (r, 0)),
            pl.BlockSpec((LORA_R, cols), lambda r: (0, 0)),
        ],
        out_specs=pl.BlockSpec((_PC, cols), lambda r: (r, 0)),
        out_shape=jax.ShapeDtypeStruct((rows, cols), _BF16),
    )(w, a, bm)


def _mlp_kernel(x_ref, xm_ref, ln_ref, wg_ref, wu_ref, wd_ref,
                a2_ref, out_ref, x1_s, h_s, acc_s):
    j = pl.program_id(1)

    @pl.when(j == 0)
    def _prep():
        x1 = x_ref[...] + xm_ref[...]
        x1_s[...] = x1
        var = jnp.mean(x1 * x1, axis=-1, keepdims=True)
        h = x1 * jax.lax.rsqrt(var + 1e-6) * ln_ref[...]
        h_s[...] = h.astype(_BF16)
        acc_s[...] = jnp.zeros_like(acc_s)

    hb = h_s[...]
    g = jnp.dot(hb, wg_ref[...], preferred_element_type=_F32)
    u = jnp.dot(hb, wu_ref[...], preferred_element_type=_F32)
    d = (g * jax.nn.sigmoid(g) + u).astype(_BF16)
    acc_s[...] += jnp.dot(d, wd_ref[...], preferred_element_type=_F32)

    @pl.when(j == _NJ - 1)
    def _fin():
        out_ref[...] = x1_s[...] + a2_ref[0, 0] * acc_s[...]


def _mlp(x2, xm2, ln2_w, wg, wu, wd, a2):
    nt = x2.shape[0] // _TB
    return pl.pallas_call(
        _mlp_kernel,
        grid=(nt, _NJ),
        in_specs=[
            pl.BlockSpec((_TB, D_MODEL), lambda t, j: (t, 0)),
            pl.BlockSpec((_TB, D_MODEL), lambda t, j: (t, 0)),
            pl.BlockSpec((1, D_MODEL), lambda t, j: (0, 0)),
            pl.BlockSpec((D_MODEL, _JB), lambda t, j: (0, j)),
            pl.BlockSpec((D_MODEL, _JB), lambda t, j: (0, j)),
            pl.BlockSpec((_JB, D_MODEL), lambda t, j: (j, 0)),
            pl.BlockSpec(memory_space=pltpu.SMEM),
        ],
        out_specs=pl.BlockSpec((_TB, D_MODEL), lambda t, j: (t, 0)),
        out_shape=jax.ShapeDtypeStruct((x2.shape[0], D_MODEL), _F32),
        scratch_shapes=[
            pltpu.VMEM((_TB, D_MODEL), _F32),
            pltpu.VMEM((_TB, D_MODEL), _BF16),
            pltpu.VMEM((_TB, D_MODEL), _F32),
        ],
    )(x2, xm2, ln2_w.reshape(1, D_MODEL), wg, wu, wd, a2)


def kernel(x_q, z_a, z_v, z_av, params):
    p = params
    x = x_q[0]

    a1 = p['alpha_1'].reshape(1, 1)
    a2 = jax.nn.sigmoid(p['alpha_2']).reshape(1, 1)
    wattn = [p['Wq_a'], p['Wk_a'], p['Wv_a'], p['Wo_a'],
             p['Wq_v'], p['Wk_v'], p['Wv_v'], p['Wo_v'],
             p['Wq_av'], p['Wk_av'], p['Wv_av'], p['Wo_av']]
    wg = _fold(p['Wg'], p['Ag'], p['Bg'])
    wu = _fold(p['Wu'], p['Au'], p['Bu'])
    wd = _fold(p['Wd'], p['Ad'], p['Bd'])

    nx, comb = _norm_router(x, p['ln1_w'], p['Wr'], p['br'], a1)
    xm = _attn(comb, nx, z_a, z_v, z_av, wattn)
    tok = B * S
    out = _mlp(x.reshape(tok, D_MODEL), xm.reshape(tok, D_MODEL),
               p['ln2_w'], wg, wu, wd, a2)
    return out.reshape(B, S, D_MODEL)


# speedup vs baseline: 2.2466x; 1.0122x over previous
import jax
import jax.numpy as jnp
from jax.experimental import pallas as pl
from jax.experimental.pallas import tpu as pltpu

D_MODEL = 1024
N_HEAD = 16
DH = 64
N_INTER = 4096
LORA_R = 8
LORA_SCALE = 2.0
N_EXPERTS = 4
B, S, L = 2, 2048, 256

_F32 = jnp.float32
_BF16 = jnp.bfloat16


def _norm_router_kernel(x_ref, ln_ref, wr_ref, br_ref, a1_ref,
                        nx_ref, comb_ref):
    x = x_ref[...]
    var = jnp.mean(x * x, axis=-1, keepdims=True)
    nx = x * jax.lax.rsqrt(var + 1e-6) * ln_ref[...][None]
    nx_ref[...] = nx.astype(_BF16)
    q_pool = jnp.mean(nx, axis=1)
    logits = jax.lax.dot_general(
        q_pool, wr_ref[...], (((1,), (0,)), ((), ())),
        preferred_element_type=_F32) + br_ref[...]
    aw = jax.nn.softmax(logits, axis=-1)
    idx = jax.lax.broadcasted_iota(jnp.int32, aw.shape, 1)
    big = jnp.int32(N_EXPERTS)
    w1 = jnp.max(aw, axis=-1, keepdims=True)
    i1 = jnp.min(jnp.where(aw >= w1, idx, big), axis=-1, keepdims=True)
    m = jnp.where(idx == i1, -jnp.inf, aw)
    w2 = jnp.max(m, axis=-1, keepdims=True)
    i2 = jnp.min(jnp.where(m >= w2, idx, big), axis=-1, keepdims=True)
    denom = w1 + w2 + 1e-10
    comb = jnp.where(idx == i1, w1, jnp.where(idx == i2, w2, 0.0)) / denom
    comb_ref[...] = comb * (1.0 / (1.0 + jnp.exp(-a1_ref[0, 0])))


def _norm_router(x, ln1_w, wr, br, a1):
    bl = x.shape[0]
    return pl.pallas_call(
        _norm_router_kernel,
        in_specs=[
            pl.BlockSpec((bl, S, D_MODEL), lambda: (0, 0, 0)),
            pl.BlockSpec((1, D_MODEL), lambda: (0, 0)),
            pl.BlockSpec((D_MODEL, N_EXPERTS), lambda: (0, 0)),
            pl.BlockSpec((1, N_EXPERTS), lambda: (0, 0)),
            pl.BlockSpec(memory_space=pltpu.SMEM),
        ],
        out_specs=(
            pl.BlockSpec((bl, S, D_MODEL), lambda: (0, 0, 0)),
            pl.BlockSpec((bl, N_EXPERTS), lambda: (0, 0)),
        ),
        out_shape=(
            jax.ShapeDtypeStruct((bl, S, D_MODEL), _BF16),
            jax.ShapeDtypeStruct((bl, N_EXPERTS), _F32),
        ),
    )(x, ln1_w.reshape(1, D_MODEL), wr, br.reshape(1, N_EXPERTS), a1)


_SB = 1024


def _attn_kernel(comb_ref, nx_ref,
                 za_ref, zv_ref, zav_ref,
                 wqa_ref, wka_ref, wva_ref, woa_ref,
                 wqv_ref, wkv_ref, wvv_ref, wov_ref,
                 wqav_ref, wkav_ref, wvav_ref, woav_ref,
                 out_ref,
                 z_st, wq_st, wk_st, wv_st, wo_st, q_s, k_s, v_s, o_s,
                 sem_z, sem_q, sem_k, sem_v, sem_o):
    b = pl.program_id(0)
    e = pl.program_id(1)

    @pl.when(e == 0)
    def _init():
        out_ref[...] = jnp.zeros_like(out_ref)

    w = comb_ref[b, e]

    @pl.when(w > 0.0)
    def _compute():
        def _start(zr, wq, wk, wv, wo):
            pltpu.make_async_copy(wq, wq_st, sem_q).start()
            pltpu.make_async_copy(zr.at[b], z_st, sem_z).start()
            pltpu.make_async_copy(wk, wk_st, sem_k).start()
            pltpu.make_async_copy(wv, wv_st, sem_v).start()
            pltpu.make_async_copy(wo, wo_st, sem_o).start()

        @pl.when(e == 0)
        def _sa():
            _start(za_ref, wqa_ref, wka_ref, wva_ref, woa_ref)

        @pl.when(e == 1)
        def _sv():
            _start(zv_ref, wqv_ref, wkv_ref, wvv_ref, wov_ref)

        @pl.when(e == 2)
        def _sav():
            _start(zav_ref, wqav_ref, wkav_ref, wvav_ref, woav_ref)

        pltpu.make_async_copy(wqa_ref, wq_st, sem_q).wait()
        wqb = wq_st[...].astype(_BF16)
        q_s[...] = jnp.dot(nx_ref[0, :_SB], wqb,
                           preferred_element_type=_F32).astype(_BF16)
        pltpu.make_async_copy(za_ref.at[b], z_st, sem_z).wait()
        z = z_st[...].astype(_BF16)
        pltpu.make_async_copy(wka_ref, wk_st, sem_k).wait()
        k_s[...] = jnp.dot(z, wk_st[...].astype(_BF16),
                           preferred_element_type=_F32).astype(_BF16)
        pltpu.make_async_copy(wva_ref, wv_st, sem_v).wait()
        v_s[...] = jnp.dot(z, wv_st[...].astype(_BF16),
                           preferred_element_type=_F32).astype(_BF16)
        pltpu.make_async_copy(woa_ref, wo_st, sem_o).wait()
        wob = wo_st[...].astype(_BF16)
        for half in range(S // _SB):
            rows_g = slice(half * _SB, (half + 1) * _SB)
            if half > 0:
                q_s[...] = jnp.dot(nx_ref[0, rows_g], wqb,
                                   preferred_element_type=_F32).astype(_BF16)
            for h in range(N_HEAD):
                cols = slice(h * DH, (h + 1) * DH)
                kh = k_s[:, cols]
                vh = v_s[:, cols]
                for sb in range(_SB // 512):
                    rows = slice(sb * 512, (sb + 1) * 512)
                    s = jax.lax.dot_general(
                        q_s[rows, cols], kh, (((1,), (1,)), ((), ())),
                        preferred_element_type=_F32) * 0.125
                    p = jax.nn.softmax(s, axis=-1).astype(_BF16)
                    o_s[rows, cols] = jnp.dot(
                        p, vh, preferred_element_type=_F32).astype(_BF16)
            out_ref[0, rows_g] += jnp.dot(
                o_s[...], wob, preferred_element_type=_F32) * w


def _attn(comb, nx, z_a, z_v, z_av, wlist):
    bl = nx.shape[0]
    anyspec = pl.BlockSpec(memory_space=pl.MemorySpace.ANY)
    return pl.pallas_call(
        _attn_kernel,
        grid=(bl, 3),
        in_specs=[
            pl.BlockSpec(memory_space=pltpu.SMEM),
            pl.BlockSpec((1, S, D_MODEL), lambda b, e: (b, 0, 0)),
        ] + [anyspec] * 15,
        out_specs=pl.BlockSpec((1, S, D_MODEL), lambda b, e: (b, 0, 0)),
        out_shape=jax.ShapeDtypeStruct((bl, S, D_MODEL), _F32),
        scratch_shapes=[
            pltpu.VMEM((L, D_MODEL), _F32),
            pltpu.VMEM((D_MODEL, D_MODEL), _F32),
            pltpu.VMEM((D_MODEL, D_MODEL), _F32),
            pltpu.VMEM((D_MODEL, D_MODEL), _F32),
            pltpu.VMEM((D_MODEL, D_MODEL), _F32),
            pltpu.VMEM((_SB, D_MODEL), _BF16),
            pltpu.VMEM((L, D_MODEL), _BF16),
            pltpu.VMEM((L, D_MODEL), _BF16),
            pltpu.VMEM((_SB, D_MODEL), _BF16),
            pltpu.SemaphoreType.DMA,
            pltpu.SemaphoreType.DMA,
            pltpu.SemaphoreType.DMA,
            pltpu.SemaphoreType.DMA,
            pltpu.SemaphoreType.DMA,
        ],
    )(comb, nx, z_a, z_v, z_av, *wlist)


_TB = 1024
_JB = 1024
_NJ = N_INTER // _JB
_PC = 512


def _fold_kernel(w_ref, a_ref, b_ref, out_ref):
    out_ref[...] = (w_ref[...] + LORA_SCALE * jnp.dot(
        a_ref[...].astype(_BF16), b_ref[...].astype(_BF16),
        preferred_element_type=_F32)).astype(_BF16)


def _fold(w, a, bm):
    rows, cols = w.shape
    return pl.pallas_call(
        _fold_kernel,
        grid=(rows // _PC,),
        in_specs=[
            pl.BlockSpec((_PC, cols), lambda r: (r, 0)),
            pl.BlockSpec((_PC, LORA_R), lambda r: (r, 0)),
            pl.BlockSpec((LORA_R, cols), lambda r: (0, 0)),
        ],
        out_specs=pl.BlockSpec((_PC, cols), lambda r: (r, 0)),
        out_shape=jax.ShapeDtypeStruct((rows, cols), _BF16),
    )(w, a, bm)


def _mlp_kernel(x_ref, xm_ref, ln_ref, wg_ref, wu_ref, wd_ref,
                a2_ref, out_ref, x1_s, h_s, acc_s):
    j = pl.program_id(1)

    @pl.when(j == 0)
    def _prep():
        x1 = x_ref[...] + xm_ref[...]
        x1_s[...] = x1
        var = jnp.mean(x1 * x1, axis=-1, keepdims=True)
        h = x1 * jax.lax.rsqrt(var + 1e-6) * ln_ref[...]
        h_s[...] = h.astype(_BF16)
        acc_s[...] = jnp.zeros_like(acc_s)

    hb = h_s[...]
    g = jnp.dot(hb, wg_ref[...], preferred_element_type=_F32)
    u = jnp.dot(hb, wu_ref[...], preferred_element_type=_F32)
    d = (g * jax.nn.sigmoid(g) + u).astype(_BF16)
    acc_s[...] += jnp.dot(d, wd_ref[...], preferred_element_type=_F32)

    @pl.when(j == _NJ - 1)
    def _fin():
        out_ref[...] = x1_s[...] + a2_ref[0, 0] * acc_s[...]


def _mlp(x2, xm2, ln2_w, wg, wu, wd, a2):
    nt = x2.shape[0] // _TB
    return pl.pallas_call(
        _mlp_kernel,
        grid=(nt, _NJ),
        in_specs=[
            pl.BlockSpec((_TB, D_MODEL), lambda t, j: (t, 0)),
            pl.BlockSpec((_TB, D_MODEL), lambda t, j: (t, 0)),
            pl.BlockSpec((1, D_MODEL), lambda t, j: (0, 0)),
            pl.BlockSpec((D_MODEL, _JB), lambda t, j: (0, j)),
            pl.BlockSpec((D_MODEL, _JB), lambda t, j: (0, j)),
            pl.BlockSpec((_JB, D_MODEL), lambda t, j: (j, 0)),
            pl.BlockSpec(memory_space=pltpu.SMEM),
        ],
        out_specs=pl.BlockSpec((_TB, D_MODEL), lambda t, j: (t, 0)),
        out_shape=jax.ShapeDtypeStruct((x2.shape[0], D_MODEL), _F32),
        scratch_shapes=[
            pltpu.VMEM((_TB, D_MODEL), _F32),
            pltpu.VMEM((_TB, D_MODEL), _BF16),
            pltpu.VMEM((_TB, D_MODEL), _F32),
        ],
    )(x2, xm2, ln2_w.reshape(1, D_MODEL), wg, wu, wd, a2)


def kernel(x_q, z_a, z_v, z_av, params):
    p = params
    x = x_q[0]

    a1 = p['alpha_1'].reshape(1, 1)
    a2 = jax.nn.sigmoid(p['alpha_2']).reshape(1, 1)
    wattn = [p['Wq_a'], p['Wk_a'], p['Wv_a'], p['Wo_a'],
             p['Wq_v'], p['Wk_v'], p['Wv_v'], p['Wo_v'],
             p['Wq_av'], p['Wk_av'], p['Wv_av'], p['Wo_av']]
    wg = _fold(p['Wg'], p['Ag'], p['Bg'])
    wu = _fold(p['Wu'], p['Au'], p['Bu'])
    wd = _fold(p['Wd'], p['Ad'], p['Bd'])

    nx, comb = _norm_router(x, p['ln1_w'], p['Wr'], p['br'], a1)
    xm = _attn(comb, nx, z_a, z_v, z_av, wattn)
    tok = B * S
    out = _mlp(x.reshape(tok, D_MODEL), xm.reshape(tok, D_MODEL),
               p['ln2_w'], wg, wu, wd, a2)
    return out.reshape(B, S, D_MODEL)


# speedup vs baseline: 2.2547x; 1.0036x over previous
import jax
import jax.numpy as jnp
from jax.experimental import pallas as pl
from jax.experimental.pallas import tpu as pltpu

D_MODEL = 1024
N_HEAD = 16
DH = 64
N_INTER = 4096
LORA_R = 8
LORA_SCALE = 2.0
N_EXPERTS = 4
B, S, L = 2, 2048, 256

_F32 = jnp.float32
_BF16 = jnp.bfloat16


def _norm_router_kernel(x_ref, ln_ref, wr_ref, br_ref, a1_ref,
                        nx_ref, comb_ref):
    x = x_ref[...]
    var = jnp.mean(x * x, axis=-1, keepdims=True)
    nx = x * jax.lax.rsqrt(var + 1e-6) * ln_ref[...][None]
    nx_ref[...] = nx.astype(_BF16)
    q_pool = jnp.mean(nx, axis=1)
    logits = jax.lax.dot_general(
        q_pool, wr_ref[...], (((1,), (0,)), ((), ())),
        preferred_element_type=_F32) + br_ref[...]
    aw = jax.nn.softmax(logits, axis=-1)
    idx = jax.lax.broadcasted_iota(jnp.int32, aw.shape, 1)
    big = jnp.int32(N_EXPERTS)
    w1 = jnp.max(aw, axis=-1, keepdims=True)
    i1 = jnp.min(jnp.where(aw >= w1, idx, big), axis=-1, keepdims=True)
    m = jnp.where(idx == i1, -jnp.inf, aw)
    w2 = jnp.max(m, axis=-1, keepdims=True)
    i2 = jnp.min(jnp.where(m >= w2, idx, big), axis=-1, keepdims=True)
    denom = w1 + w2 + 1e-10
    comb = jnp.where(idx == i1, w1, jnp.where(idx == i2, w2, 0.0)) / denom
    comb_ref[...] = comb * (1.0 / (1.0 + jnp.exp(-a1_ref[0, 0])))


def _norm_router(x, ln1_w, wr, br, a1):
    bl = x.shape[0]
    return pl.pallas_call(
        _norm_router_kernel,
        in_specs=[
            pl.BlockSpec((bl, S, D_MODEL), lambda: (0, 0, 0)),
            pl.BlockSpec((1, D_MODEL), lambda: (0, 0)),
            pl.BlockSpec((D_MODEL, N_EXPERTS), lambda: (0, 0)),
            pl.BlockSpec((1, N_EXPERTS), lambda: (0, 0)),
            pl.BlockSpec(memory_space=pltpu.SMEM),
        ],
        out_specs=(
            pl.BlockSpec((bl, S, D_MODEL), lambda: (0, 0, 0)),
            pl.BlockSpec((bl, N_EXPERTS), lambda: (0, 0)),
        ),
        out_shape=(
            jax.ShapeDtypeStruct((bl, S, D_MODEL), _BF16),
            jax.ShapeDtypeStruct((bl, N_EXPERTS), _F32),
        ),
    )(x, ln1_w.reshape(1, D_MODEL), wr, br.reshape(1, N_EXPERTS), a1)


_SB = 1024


def _attn_kernel(comb_ref, nx_ref,
                 za_ref, zv_ref, zav_ref,
                 wqa_ref, wka_ref, wva_ref, woa_ref,
                 wqv_ref, wkv_ref, wvv_ref, wov_ref,
                 wqav_ref, wkav_ref, wvav_ref, woav_ref,
                 out_ref,
                 z_st, wq_st, wk_st, wv_st, wo_st, q_s, k_s, v_s, o_s,
                 acc_s, sem_z, sem_q, sem_k, sem_v, sem_o):
    b = pl.program_id(0)
    e = pl.program_id(1)

    @pl.when(e == 0)
    def _init():
        acc_s[...] = jnp.zeros_like(acc_s)

    w = comb_ref[b, e]

    @pl.when(w > 0.0)
    def _compute():
        def _start(zr, wq, wk, wv, wo):
            pltpu.make_async_copy(wq, wq_st, sem_q).start()
            pltpu.make_async_copy(zr.at[b], z_st, sem_z).start()
            pltpu.make_async_copy(wk, wk_st, sem_k).start()
            pltpu.make_async_copy(wv, wv_st, sem_v).start()
            pltpu.make_async_copy(wo, wo_st, sem_o).start()

        @pl.when(e == 0)
        def _sa():
            _start(za_ref, wqa_ref, wka_ref, wva_ref, woa_ref)

        @pl.when(e == 1)
        def _sv():
            _start(zv_ref, wqv_ref, wkv_ref, wvv_ref, wov_ref)

        @pl.when(e == 2)
        def _sav():
            _start(zav_ref, wqav_ref, wkav_ref, wvav_ref, woav_ref)

        pltpu.make_async_copy(wqa_ref, wq_st, sem_q).wait()
        wqb = wq_st[...].astype(_BF16)
        q_s[...] = jnp.dot(nx_ref[0, :_SB], wqb,
                           preferred_element_type=_F32).astype(_BF16)
        pltpu.make_async_copy(za_ref.at[b], z_st, sem_z).wait()
        z = z_st[...].astype(_BF16)
        pltpu.make_async_copy(wka_ref, wk_st, sem_k).wait()
        k_s[...] = jnp.dot(z, wk_st[...].astype(_BF16),
                           preferred_element_type=_F32).astype(_BF16)
        pltpu.make_async_copy(wva_ref, wv_st, sem_v).wait()
        v_s[...] = jnp.dot(z, wv_st[...].astype(_BF16),
                           preferred_element_type=_F32).astype(_BF16)
        pltpu.make_async_copy(woa_ref, wo_st, sem_o).wait()
        wob = wo_st[...].astype(_BF16)
        for half in range(S // _SB):
            rows_g = slice(half * _SB, (half + 1) * _SB)
            if half > 0:
                q_s[...] = jnp.dot(nx_ref[0, rows_g], wqb,
                                   preferred_element_type=_F32).astype(_BF16)
            for h in range(N_HEAD):
                cols = slice(h * DH, (h + 1) * DH)
                kh = k_s[:, cols]
                vh = v_s[:, cols]
                for sb in range(_SB // 512):
                    rows = slice(sb * 512, (sb + 1) * 512)
                    s = jax.lax.dot_general(
                        q_s[rows, cols], kh, (((1,), (1,)), ((), ())),
                        preferred_element_type=_F32) * 0.125
                    p = jax.nn.softmax(s, axis=-1).astype(_BF16)
                    o_s[rows, cols] = jnp.dot(
                        p, vh, preferred_element_type=_F32).astype(_BF16)
            acc_s[rows_g] += jnp.dot(
                o_s[...], wob, preferred_element_type=_F32) * w

    @pl.when(e == 2)
    def _fin():
        out_ref[0] = acc_s[...].astype(_BF16)


def _attn(comb, nx, z_a, z_v, z_av, wlist):
    bl = nx.shape[0]
    anyspec = pl.BlockSpec(memory_space=pl.MemorySpace.ANY)
    return pl.pallas_call(
        _attn_kernel,
        grid=(bl, 3),
        in_specs=[
            pl.BlockSpec(memory_space=pltpu.SMEM),
            pl.BlockSpec((1, S, D_MODEL), lambda b, e: (b, 0, 0)),
        ] + [anyspec] * 15,
        out_specs=pl.BlockSpec((1, S, D_MODEL), lambda b, e: (b, 0, 0)),
        out_shape=jax.ShapeDtypeStruct((bl, S, D_MODEL), _BF16),
        scratch_shapes=[
            pltpu.VMEM((L, D_MODEL), _F32),
            pltpu.VMEM((D_MODEL, D_MODEL), _F32),
            pltpu.VMEM((D_MODEL, D_MODEL), _F32),
            pltpu.VMEM((D_MODEL, D_MODEL), _F32),
            pltpu.VMEM((D_MODEL, D_MODEL), _F32),
            pltpu.VMEM((_SB, D_MODEL), _BF16),
            pltpu.VMEM((L, D_MODEL), _BF16),
            pltpu.VMEM((L, D_MODEL), _BF16),
            pltpu.VMEM((_SB, D_MODEL), _BF16),
            pltpu.VMEM((S, D_MODEL), _F32),
            pltpu.SemaphoreType.DMA,
            pltpu.SemaphoreType.DMA,
            pltpu.SemaphoreType.DMA,
            pltpu.SemaphoreType.DMA,
            pltpu.SemaphoreType.DMA,
        ],
    )(comb, nx, z_a, z_v, z_av, *wlist)


_TB = 1024
_JB = 1024
_NJ = N_INTER // _JB
_PC = 512


def _fold_kernel(w_ref, a_ref, b_ref, out_ref):
    out_ref[...] = (w_ref[...] + LORA_SCALE * jnp.dot(
        a_ref[...].astype(_BF16), b_ref[...].astype(_BF16),
        preferred_element_type=_F32)).astype(_BF16)


def _fold(w, a, bm):
    rows, cols = w.shape
    return pl.pallas_call(
        _fold_kernel,
        grid=(rows // _PC,),
        in_specs=[
            pl.BlockSpec((_PC, cols), lambda r: (r, 0)),
            pl.BlockSpec((_PC, LORA_R), lambda r: (r, 0)),
            pl.BlockSpec((LORA_R, cols), lambda r: (0, 0)),
        ],
        out_specs=pl.BlockSpec((_PC, cols), lambda r: (r, 0)),
        out_shape=jax.ShapeDtypeStruct((rows, cols), _BF16),
    )(w, a, bm)


def _mlp_kernel(x_ref, xm_ref, ln_ref, wg_ref, wu_ref, wd_ref,
                a2_ref, out_ref, x1_s, h_s, acc_s):
    j = pl.program_id(1)

    @pl.when(j == 0)
    def _prep():
        x1 = x_ref[...] + xm_ref[...].astype(_F32)
        x1_s[...] = x1
        var = jnp.mean(x1 * x1, axis=-1, keepdims=True)
        h = x1 * jax.lax.rsqrt(var + 1e-6) * ln_ref[...]
        h_s[...] = h.astype(_BF16)
        acc_s[...] = jnp.zeros_like(acc_s)

    hb = h_s[...]
    g = jnp.dot(hb, wg_ref[...], preferred_element_type=_F32)
    u = jnp.dot(hb, wu_ref[...], preferred_element_type=_F32)
    d = (g * jax.nn.sigmoid(g) + u).astype(_BF16)
    acc_s[...] += jnp.dot(d, wd_ref[...], preferred_element_type=_F32)

    @pl.when(j == _NJ - 1)
    def _fin():
        out_ref[...] = x1_s[...] + a2_ref[0, 0] * acc_s[...]


def _mlp(x2, xm2, ln2_w, wg, wu, wd, a2):
    nt = x2.shape[0] // _TB
    return pl.pallas_call(
        _mlp_kernel,
        grid=(nt, _NJ),
        in_specs=[
            pl.BlockSpec((_TB, D_MODEL), lambda t, j: (t, 0)),
            pl.BlockSpec((_TB, D_MODEL), lambda t, j: (t, 0)),
            pl.BlockSpec((1, D_MODEL), lambda t, j: (0, 0)),
            pl.BlockSpec((D_MODEL, _JB), lambda t, j: (0, j)),
            pl.BlockSpec((D_MODEL, _JB), lambda t, j: (0, j)),
            pl.BlockSpec((_JB, D_MODEL), lambda t, j: (j, 0)),
            pl.BlockSpec(memory_space=pltpu.SMEM),
        ],
        out_specs=pl.BlockSpec((_TB, D_MODEL), lambda t, j: (t, 0)),
        out_shape=jax.ShapeDtypeStruct((x2.shape[0], D_MODEL), _F32),
        scratch_shapes=[
            pltpu.VMEM((_TB, D_MODEL), _F32),
            pltpu.VMEM((_TB, D_MODEL), _BF16),
            pltpu.VMEM((_TB, D_MODEL), _F32),
        ],
    )(x2, xm2, ln2_w.reshape(1, D_MODEL), wg, wu, wd, a2)


def kernel(x_q, z_a, z_v, z_av, params):
    p = params
    x = x_q[0]

    a1 = p['alpha_1'].reshape(1, 1)
    a2 = jax.nn.sigmoid(p['alpha_2']).reshape(1, 1)
    wattn = [p['Wq_a'], p['Wk_a'], p['Wv_a'], p['Wo_a'],
             p['Wq_v'], p['Wk_v'], p['Wv_v'], p['Wo_v'],
             p['Wq_av'], p['Wk_av'], p['Wv_av'], p['Wo_av']]
    wg = _fold(p['Wg'], p['Ag'], p['Bg'])
    wu = _fold(p['Wu'], p['Au'], p['Bu'])
    wd = _fold(p['Wd'], p['Ad'], p['Bd'])

    nx, comb = _norm_router(x, p['ln1_w'], p['Wr'], p['br'], a1)
    xm = _attn(comb, nx, z_a, z_v, z_av, wattn)
    tok = B * S
    out = _mlp(x.reshape(tok, D_MODEL), xm.reshape(tok, D_MODEL),
               p['ln2_w'], wg, wu, wd, a2)
    return out.reshape(B, S, D_MODEL)
